# baseline (device time: 70583 ns/iter reference)
import jax
import jax.numpy as jnp
from jax import lax
from jax.experimental import pallas as pl
from jax.experimental.pallas import tpu as pltpu

N_DEV = 4
M = 256
H = M // 2
N_PER = 4096


def kernel(x, W):
    def body(x_ref, w_ref, out_ref, gather_ref, stats_ref,
             r_send, r_recv, l_send, l_recv, st_send, st_recv):
        my = lax.axis_index("i")
        left = (my - 1) % N_DEV
        right = (my + 1) % N_DEV

        logits = jnp.dot(
            x_ref[...].astype(jnp.bfloat16),
            w_ref[...].astype(jnp.bfloat16),
            preferred_element_type=jnp.float32,
        )
        gather_ref[pl.ds(my * M, M), :] = logits.astype(jnp.bfloat16)

        barrier_sem = pltpu.get_barrier_semaphore()
        for d in (1, 2, 3):
            pl.semaphore_signal(
                barrier_sem, inc=1,
                device_id=((my + d) % N_DEV,),
                device_id_type=pl.DeviceIdType.MESH,
            )
        pl.semaphore_wait(barrier_sem, 3)

        data_sends = []
        c = None
        for h in range(N_DEV - 1):
            o_r = (my - h) % N_DEV
            o_l = (my + h) % N_DEV
            rr = pltpu.make_async_remote_copy(
                src_ref=gather_ref.at[pl.ds(o_r * M, H), :],
                dst_ref=gather_ref.at[pl.ds(o_r * M, H), :],
                send_sem=r_send.at[h],
                recv_sem=r_recv.at[h],
                device_id=(right,),
                device_id_type=pl.DeviceIdType.MESH,
            )
            rl = pltpu.make_async_remote_copy(
                src_ref=gather_ref.at[pl.ds(o_l * M + H, H), :],
                dst_ref=gather_ref.at[pl.ds(o_l * M + H, H), :],
                send_sem=l_send.at[h],
                recv_sem=l_recv.at[h],
                device_id=(left,),
                device_id_type=pl.DeviceIdType.MESH,
            )
            rr.start()
            rl.start()
            data_sends += [rr, rl]

            if h == 0:
                m0 = logits.max(-1, keepdims=True)
                s0 = jnp.exp(logits - m0).sum(-1, keepdims=True)
                stats_ref[pl.ds(my * M, M), 0:1] = m0
                stats_ref[pl.ds(my * M, M), 1:2] = s0
                st_sends = []
                for d in (1, 2, 3):
                    r = pltpu.make_async_remote_copy(
                        src_ref=stats_ref.at[pl.ds(my * M, M), :],
                        dst_ref=stats_ref.at[pl.ds(my * M, M), :],
                        send_sem=st_send.at[d - 1],
                        recv_sem=st_recv.at[d - 1],
                        device_id=((my + d) % N_DEV,),
                        device_id_type=pl.DeviceIdType.MESH,
                    )
                    r.start()
                    st_sends.append(r)
                st_recvs = []
                for d in (1, 2, 3):
                    sender = (my - d) % N_DEV
                    st_recvs.append(pltpu.make_async_remote_copy(
                        src_ref=stats_ref.at[pl.ds(my * M, M), :],
                        dst_ref=stats_ref.at[pl.ds(sender * M, M), :],
                        send_sem=st_send.at[d - 1],
                        recv_sem=st_recv.at[d - 1],
                        device_id=(sender,),
                        device_id_type=pl.DeviceIdType.MESH,
                    ))
                for r in st_recvs:
                    r.wait_recv()
                mk = [stats_ref[pl.ds(k * M, M), 0:1] for k in range(N_DEV)]
                sk = [stats_ref[pl.ds(k * M, M), 1:2] for k in range(N_DEV)]
                mm = mk[0]
                for v in mk[1:]:
                    mm = jnp.maximum(mm, v)
                z = jnp.zeros((M, 1), jnp.float32)
                for mv, sv in zip(mk, sk):
                    z = z + sv * jnp.exp(mv - mm)
                c = mm + jnp.log(z)

            out_ref[0:H, pl.ds(o_r * N_PER, N_PER)] = jnp.exp(
                gather_ref[pl.ds(o_r * M, H), :].astype(jnp.float32) - c[0:H]
            )
            out_ref[H:M, pl.ds(o_l * N_PER, N_PER)] = jnp.exp(
                gather_ref[pl.ds(o_l * M + H, H), :].astype(jnp.float32)
                - c[H:M]
            )
            rr.wait_recv()
            rl.wait_recv()

        o_rt = (my + 1) % N_DEV
        o_lt = (my - 1) % N_DEV
        out_ref[0:H, pl.ds(o_rt * N_PER, N_PER)] = jnp.exp(
            gather_ref[pl.ds(o_rt * M, H), :].astype(jnp.float32) - c[0:H]
        )
        out_ref[H:M, pl.ds(o_lt * N_PER, N_PER)] = jnp.exp(
            gather_ref[pl.ds(o_lt * M + H, H), :].astype(jnp.float32) - c[H:M]
        )

        for r in data_sends + st_sends:
            r.wait_send()

    return pl.pallas_call(
        body,
        out_shape=jax.ShapeDtypeStruct((M, N_DEV * N_PER), jnp.float32),
        in_specs=[
            pl.BlockSpec(memory_space=pltpu.VMEM),
            pl.BlockSpec(memory_space=pltpu.VMEM),
        ],
        out_specs=pl.BlockSpec(memory_space=pltpu.VMEM),
        scratch_shapes=[
            pltpu.VMEM((N_DEV * M, N_PER), jnp.bfloat16),
            pltpu.VMEM((N_DEV * M, 8), jnp.float32),
            pltpu.SemaphoreType.DMA((N_DEV - 1,)),
            pltpu.SemaphoreType.DMA((N_DEV - 1,)),
            pltpu.SemaphoreType.DMA((N_DEV - 1,)),
            pltpu.SemaphoreType.DMA((N_DEV - 1,)),
            pltpu.SemaphoreType.DMA((N_DEV - 1,)),
            pltpu.SemaphoreType.DMA((N_DEV - 1,)),
        ],
        compiler_params=pltpu.CompilerParams(collective_id=0),
    )(x, W)


# device time: 67828 ns/iter; 1.0406x vs baseline; 1.0406x over previous
import jax
import jax.numpy as jnp
from jax import lax
from jax.experimental import pallas as pl
from jax.experimental.pallas import tpu as pltpu

N_DEV = 4
M = 256
H = M // 2
Q = H // 2
N_PER = 4096


def kernel(x, W):
    def body(x_ref, w_ref, out_ref, gather_ref, stats_ref,
             r_send, r_recv, l_send, l_recv, st_send, st_recv):
        my = lax.axis_index("i")
        left = (my - 1) % N_DEV
        right = (my + 1) % N_DEV

        logits = jnp.dot(
            x_ref[...].astype(jnp.bfloat16),
            w_ref[...].astype(jnp.bfloat16),
            preferred_element_type=jnp.float32,
        )
        gather_ref[pl.ds(my * M, M), :] = logits.astype(jnp.bfloat16)

        barrier_sem = pltpu.get_barrier_semaphore()
        for d in (1, 2, 3):
            pl.semaphore_signal(
                barrier_sem, inc=1,
                device_id=((my + d) % N_DEV,),
                device_id_type=pl.DeviceIdType.MESH,
            )
        pl.semaphore_wait(barrier_sem, 3)

        data_sends = []
        rr = {}
        rl = {}
        c = None
        for h in range(N_DEV - 1):
            o_r = (my - h) % N_DEV
            o_l = (my + h) % N_DEV
            for sub in range(2):
                if h > 0:
                    rr[(h - 1, sub)].wait_recv()
                    rl[(h - 1, sub)].wait_recv()
                r = pltpu.make_async_remote_copy(
                    src_ref=gather_ref.at[pl.ds(o_r * M + sub * Q, Q), :],
                    dst_ref=gather_ref.at[pl.ds(o_r * M + sub * Q, Q), :],
                    send_sem=r_send.at[h * 2 + sub],
                    recv_sem=r_recv.at[h * 2 + sub],
                    device_id=(right,),
                    device_id_type=pl.DeviceIdType.MESH,
                )
                l = pltpu.make_async_remote_copy(
                    src_ref=gather_ref.at[pl.ds(o_l * M + H + sub * Q, Q), :],
                    dst_ref=gather_ref.at[pl.ds(o_l * M + H + sub * Q, Q), :],
                    send_sem=l_send.at[h * 2 + sub],
                    recv_sem=l_recv.at[h * 2 + sub],
                    device_id=(left,),
                    device_id_type=pl.DeviceIdType.MESH,
                )
                r.start()
                l.start()
                rr[(h, sub)] = r
                rl[(h, sub)] = l
                data_sends += [r, l]

            if h == 0:
                m0 = logits.max(-1, keepdims=True)
                s0 = jnp.exp(logits - m0).sum(-1, keepdims=True)
                stats_ref[pl.ds(my * M, M), 0:1] = m0
                stats_ref[pl.ds(my * M, M), 1:2] = s0
                st_sends = []
                for d in (1, 2, 3):
                    r = pltpu.make_async_remote_copy(
                        src_ref=stats_ref.at[pl.ds(my * M, M), :],
                        dst_ref=stats_ref.at[pl.ds(my * M, M), :],
                        send_sem=st_send.at[d - 1],
                        recv_sem=st_recv.at[d - 1],
                        device_id=((my + d) % N_DEV,),
                        device_id_type=pl.DeviceIdType.MESH,
                    )
                    r.start()
                    st_sends.append(r)
                st_recvs = []
                for d in (1, 2, 3):
                    sender = (my - d) % N_DEV
                    st_recvs.append(pltpu.make_async_remote_copy(
                        src_ref=stats_ref.at[pl.ds(my * M, M), :],
                        dst_ref=stats_ref.at[pl.ds(sender * M, M), :],
                        send_sem=st_send.at[d - 1],
                        recv_sem=st_recv.at[d - 1],
                        device_id=(sender,),
                        device_id_type=pl.DeviceIdType.MESH,
                    ))
                for r in st_recvs:
                    r.wait_recv()
                mk = [stats_ref[pl.ds(k * M, M), 0:1] for k in range(N_DEV)]
                sk = [stats_ref[pl.ds(k * M, M), 1:2] for k in range(N_DEV)]
                mm = mk[0]
                for v in mk[1:]:
                    mm = jnp.maximum(mm, v)
                z = jnp.zeros((M, 1), jnp.float32)
                for mv, sv in zip(mk, sk):
                    z = z + sv * jnp.exp(mv - mm)
                c = mm + jnp.log(z)

            out_ref[0:H, pl.ds(o_r * N_PER, N_PER)] = jnp.exp(
                gather_ref[pl.ds(o_r * M, H), :].astype(jnp.float32) - c[0:H]
            )
            out_ref[H:M, pl.ds(o_l * N_PER, N_PER)] = jnp.exp(
                gather_ref[pl.ds(o_l * M + H, H), :].astype(jnp.float32)
                - c[H:M]
            )

        o_rt = (my + 1) % N_DEV
        o_lt = (my - 1) % N_DEV
        last = N_DEV - 2
        for sub in range(2):
            r0 = sub * Q
            rr[(last, sub)].wait_recv()
            out_ref[r0:r0 + Q, pl.ds(o_rt * N_PER, N_PER)] = jnp.exp(
                gather_ref[pl.ds(o_rt * M + r0, Q), :].astype(jnp.float32)
                - c[r0:r0 + Q]
            )
            rl[(last, sub)].wait_recv()
            out_ref[H + r0:H + r0 + Q, pl.ds(o_lt * N_PER, N_PER)] = jnp.exp(
                gather_ref[pl.ds(o_lt * M + H + r0, Q), :].astype(jnp.float32)
                - c[H + r0:H + r0 + Q]
            )

        for r in data_sends + st_sends:
            r.wait_send()

    return pl.pallas_call(
        body,
        out_shape=jax.ShapeDtypeStruct((M, N_DEV * N_PER), jnp.float32),
        in_specs=[
            pl.BlockSpec(memory_space=pltpu.VMEM),
            pl.BlockSpec(memory_space=pltpu.VMEM),
        ],
        out_specs=pl.BlockSpec(memory_space=pltpu.VMEM),
        scratch_shapes=[
            pltpu.VMEM((N_DEV * M, N_PER), jnp.bfloat16),
            pltpu.VMEM((N_DEV * M, 8), jnp.float32),
            pltpu.SemaphoreType.DMA(((N_DEV - 1) * 2,)),
            pltpu.SemaphoreType.DMA(((N_DEV - 1) * 2,)),
            pltpu.SemaphoreType.DMA(((N_DEV - 1) * 2,)),
            pltpu.SemaphoreType.DMA(((N_DEV - 1) * 2,)),
            pltpu.SemaphoreType.DMA((N_DEV - 1,)),
            pltpu.SemaphoreType.DMA((N_DEV - 1,)),
        ],
        compiler_params=pltpu.CompilerParams(collective_id=0),
    )(x, W)
